# baseline (speedup 1.0000x reference)
import jax
import jax.numpy as jnp
from jax import lax
from jax.experimental import pallas as pl
from jax.experimental.pallas import tpu as pltpu
from jax.experimental.pallas import tpu_sc as plsc

N_NODES = 10000
N_EDGES = 320000
EMB = 64

BN = 2000
BE = 4000

NCHUNK = 2
E_CHUNK = N_EDGES // NCHUNK

NUM_CORES = 2
NUM_TILES = 16
NW = NUM_CORES * NUM_TILES
E_PER_W = E_CHUNK // NW
KGG = 200
KS = 200
N_PAD = 10240
ROWS_PER_TILE = N_PAD // NUM_TILES


def _gather_body(table_hbm, idx_hbm, out_hbm, idx_v, rows_v, sem):
    c = lax.axis_index("c")
    s = lax.axis_index("s")
    base = (c * NUM_TILES + s) * E_PER_W

    def chunk(j, carry):
        off = base + j * KGG
        pltpu.sync_copy(idx_hbm.at[pl.ds(off, KGG)], idx_v)
        pltpu.async_copy(table_hbm.at[idx_v], rows_v, sem).wait()
        pltpu.sync_copy(rows_v, out_hbm.at[pl.ds(off, KGG)])
        return carry

    lax.fori_loop(0, E_PER_W // KGG, chunk, 0)


_gather = pl.kernel(
    _gather_body,
    out_type=jax.ShapeDtypeStruct((E_CHUNK, 2 * EMB), jnp.float32),
    mesh=plsc.VectorSubcoreMesh(core_axis_name="c", subcore_axis_name="s"),
    scratch_types=[
        pltpu.VMEM((KGG,), jnp.int32),
        pltpu.VMEM((KGG, 2 * EMB), jnp.float32),
        pltpu.SemaphoreType.DMA,
    ],
)


def _scatter_body(h_hbm, idx_hbm, out_hbm, idx_v, rows_v, acc_sh, sem):
    c = lax.axis_index("c")
    s = lax.axis_index("s")

    zvec = jnp.zeros((16,), jnp.float32)

    def zrow(i, carry):
        for j in range(2 * EMB // 16):
            rows_v[i, pl.ds(j * 16, 16)] = zvec
        return carry

    lax.fori_loop(0, KS, zrow, 0)
    for t in range(ROWS_PER_TILE // KS):
        pltpu.sync_copy(rows_v, acc_sh.at[pl.ds(s * ROWS_PER_TILE + t * KS, KS)])
    rem = ROWS_PER_TILE - (ROWS_PER_TILE // KS) * KS
    if rem:
        pltpu.sync_copy(
            rows_v.at[pl.ds(0, rem)],
            acc_sh.at[pl.ds(s * ROWS_PER_TILE + (ROWS_PER_TILE // KS) * KS, rem)],
        )
    plsc.subcore_barrier()

    base = (c * NUM_TILES + s) * E_PER_W

    def chunk(j, carry):
        off = base + j * KS
        pltpu.sync_copy(idx_hbm.at[pl.ds(off, KS)], idx_v)
        pltpu.sync_copy(h_hbm.at[pl.ds(off, KS)], rows_v)
        pltpu.sync_copy(rows_v, acc_sh.at[idx_v], add=True)
        return carry

    lax.fori_loop(0, E_PER_W // KS, chunk, 0)
    plsc.subcore_barrier()
    pltpu.sync_copy(
        acc_sh.at[pl.ds(s * ROWS_PER_TILE, ROWS_PER_TILE)],
        out_hbm.at[c, pl.ds(s * ROWS_PER_TILE, ROWS_PER_TILE)],
    )


_scatter = pl.kernel(
    _scatter_body,
    out_type=jax.ShapeDtypeStruct((NUM_CORES, N_PAD, 2 * EMB), jnp.float32),
    mesh=plsc.VectorSubcoreMesh(core_axis_name="c", subcore_axis_name="s"),
    scratch_types=[
        pltpu.VMEM((KS,), jnp.int32),
        pltpu.VMEM((KS, 2 * EMB), jnp.float32),
        pltpu.VMEM_SHARED((N_PAD, 2 * EMB), jnp.float32),
        pltpu.SemaphoreType.DMA,
    ],
)


def _edge_mlp_body(xs_ref, ea_ref, w1at_ref, w1ab_ref, b1a_ref, w1bx_ref, b1bx_ref, h_ref):
    h1 = jnp.maximum(
        jnp.dot(xs_ref[...], w1at_ref[...], preferred_element_type=jnp.float32)
        + jnp.dot(ea_ref[...], w1ab_ref[...], preferred_element_type=jnp.float32)
        + b1a_ref[...],
        0.0,
    )
    h_ref[...] = jnp.maximum(
        jnp.dot(h1, w1bx_ref[...], preferred_element_type=jnp.float32) + b1bx_ref[...],
        0.0,
    )


def _node_mlp_body(x_ref, *refs):
    agg_refs = refs[:NCHUNK]
    w2at_ref, w2abx_ref, b2a_ref, w2b_ref, b2b_ref, out_ref = refs[NCHUNK:]
    agg = agg_refs[0][0] + agg_refs[0][1]
    for k in range(1, NCHUNK):
        agg = agg + agg_refs[k][0] + agg_refs[k][1]
    h = jnp.maximum(
        jnp.dot(x_ref[...], w2at_ref[...], preferred_element_type=jnp.float32)
        + jnp.dot(agg, w2abx_ref[...], preferred_element_type=jnp.float32)
        + b2a_ref[...],
        0.0,
    )
    out_ref[...] = jnp.maximum(
        jnp.dot(h, w2b_ref[...], preferred_element_type=jnp.float32) + b2b_ref[...],
        0.0,
    )


def _full_spec(shape):
    return pl.BlockSpec(shape, lambda i: (0,) * len(shape))


def kernel(x, edge_index, edge_attr, u, batch, W1a, b1a, W1b, b1b, W2a, b2a, W2b, b2b):
    senders = edge_index[0]
    receivers = edge_index[1]
    w1a_top, w1a_bot = W1a[:2 * EMB], W1a[2 * EMB:]
    w2a_top, w2a_bot = W2a[:2 * EMB], W2a[2 * EMB:]
    w1b_x = jnp.concatenate([W1b, jnp.zeros((EMB, EMB), jnp.float32)], axis=1)
    b1b_x = jnp.concatenate([b1b, jnp.zeros((EMB,), jnp.float32)]).reshape(1, 2 * EMB)
    w2ab_x = jnp.concatenate([w2a_bot, jnp.zeros((EMB, EMB), jnp.float32)], axis=0)
    b1a2 = b1a.reshape(1, EMB)
    b2a2 = b2a.reshape(1, EMB)
    b2b2 = b2b.reshape(1, EMB)

    def edge_mlp(xs_c, ea_c):
        return pl.pallas_call(
            _edge_mlp_body,
            grid=(E_CHUNK // BE,),
            in_specs=[
                pl.BlockSpec((BE, 2 * EMB), lambda i: (i, 0)),
                pl.BlockSpec((BE, EMB), lambda i: (i, 0)),
                _full_spec((2 * EMB, EMB)),
                _full_spec((EMB, EMB)),
                _full_spec((1, EMB)),
                _full_spec((EMB, 2 * EMB)),
                _full_spec((1, 2 * EMB)),
            ],
            out_specs=pl.BlockSpec((BE, 2 * EMB), lambda i: (i, 0)),
            out_shape=jax.ShapeDtypeStruct((E_CHUNK, 2 * EMB), jnp.float32),
        )(xs_c, ea_c, w1a_top, w1a_bot, b1a2, w1b_x, b1b_x)

    partials = []
    for k in range(NCHUNK):
        lo = k * E_CHUNK
        xs_k = _gather(x, lax.dynamic_slice_in_dim(senders, lo, E_CHUNK))
        h_k = edge_mlp(xs_k, lax.dynamic_slice_in_dim(edge_attr, lo, E_CHUNK, 0))
        partials.append(_scatter(h_k, lax.dynamic_slice_in_dim(receivers, lo, E_CHUNK)))

    out = pl.pallas_call(
        _node_mlp_body,
        grid=(N_NODES // BN,),
        in_specs=[
            pl.BlockSpec((BN, 2 * EMB), lambda i: (i, 0)),
        ]
        + [
            pl.BlockSpec((NUM_CORES, BN, 2 * EMB), lambda i: (0, i, 0))
            for _ in range(NCHUNK)
        ]
        + [
            _full_spec((2 * EMB, EMB)),
            _full_spec((2 * EMB, EMB)),
            _full_spec((1, EMB)),
            _full_spec((EMB, EMB)),
            _full_spec((1, EMB)),
        ],
        out_specs=pl.BlockSpec((BN, EMB), lambda i: (i, 0)),
        out_shape=jax.ShapeDtypeStruct((N_NODES, EMB), jnp.float32),
    )(x, *partials, w2a_top, w2ab_x, b2a2, W2b, b2b2)
    return out


# speedup vs baseline: 1.1965x; 1.1965x over previous
import jax
import jax.numpy as jnp
from jax import lax
from jax.experimental import pallas as pl
from jax.experimental.pallas import tpu as pltpu
from jax.experimental.pallas import tpu_sc as plsc

N_NODES = 10000
N_EDGES = 320000
EMB = 64

BN = 2000
BE = 4000

NUM_CORES = 2
NUM_TILES = 16
NW = NUM_CORES * NUM_TILES
E_PER_W = N_EDGES // NW
KGG = 200
NCHG = E_PER_W // KGG
KS = 200
NCHS = E_PER_W // KS
N_PAD = 10240
ROWS_PER_TILE = N_PAD // NUM_TILES


def _gather_body(table_hbm, idx_hbm, out_hbm, idx_v, rows_v, gsem, w0sem, w1sem):
    c = lax.axis_index("c")
    s = lax.axis_index("s")
    wid = c * NUM_TILES + s
    base = wid * E_PER_W
    pltpu.sync_copy(idx_hbm.at[pl.ds(base, E_PER_W)], idx_v)

    wsems = (w0sem, w1sem)

    def outer(g, carry):
        for b in range(2):
            j = g * 2 + b

            @pl.when(g > 0)
            def _():
                pltpu.make_async_copy(
                    rows_v.at[b], out_hbm.at[pl.ds(base, KGG)], wsems[b]
                ).wait()

            pltpu.async_copy(table_hbm.at[idx_v.at[pl.ds(j * KGG, KGG)]], rows_v.at[b], gsem).wait()
            pltpu.async_copy(
                rows_v.at[b], out_hbm.at[pl.ds(base + j * KGG, KGG)], wsems[b]
            )
        return carry

    lax.fori_loop(0, NCHG // 2, outer, 0)
    for b in range(2):
        pltpu.make_async_copy(
            rows_v.at[b], out_hbm.at[pl.ds(base, KGG)], wsems[b]
        ).wait()


_gather = pl.kernel(
    _gather_body,
    out_type=jax.ShapeDtypeStruct((N_EDGES, 2 * EMB), jnp.float32),
    mesh=plsc.VectorSubcoreMesh(core_axis_name="c", subcore_axis_name="s"),
    scratch_types=[
        pltpu.VMEM((E_PER_W,), jnp.int32),
        pltpu.VMEM((2, KGG, 2 * EMB), jnp.float32),
        pltpu.SemaphoreType.DMA,
        pltpu.SemaphoreType.DMA,
        pltpu.SemaphoreType.DMA,
    ],
)


def _scatter_body(h_hbm, idx_hbm, out_hbm, idx_v, rows_v, acc_sh, hsem0, hsem1):
    c = lax.axis_index("c")
    s = lax.axis_index("s")
    wid = c * NUM_TILES + s
    base = wid * E_PER_W

    zvec = jnp.zeros((16,), jnp.float32)

    def zrow(i, carry):
        for j in range(EMB // 16):
            rows_v[0, i, pl.ds(j * 16, 16)] = zvec
        return carry

    lax.fori_loop(0, KS, zrow, 0)
    for t in range(ROWS_PER_TILE // KS):
        pltpu.sync_copy(rows_v.at[0], acc_sh.at[pl.ds(s * ROWS_PER_TILE + t * KS, KS)])
    rem = ROWS_PER_TILE - (ROWS_PER_TILE // KS) * KS
    if rem:
        pltpu.sync_copy(
            rows_v.at[0, pl.ds(0, rem)],
            acc_sh.at[pl.ds(s * ROWS_PER_TILE + (ROWS_PER_TILE // KS) * KS, rem)],
        )
    pltpu.sync_copy(idx_hbm.at[pl.ds(wid * NCHS, NCHS)], idx_v)
    plsc.subcore_barrier()

    hsems = (hsem0, hsem1)
    pltpu.async_copy(
        h_hbm.at[pl.ds(base, KS), pl.ds(0, EMB)], rows_v.at[0], hsems[0]
    )

    def outer(g, carry):
        for b in range(2):
            j = g * 2 + b
            nb = 1 - b
            pltpu.make_async_copy(
                h_hbm.at[pl.ds(base, KS), pl.ds(0, EMB)], rows_v.at[b], hsems[b]
            ).wait()

            @pl.when(j + 1 < NCHS)
            def _():
                pltpu.async_copy(
                    h_hbm.at[pl.ds(base + (j + 1) * KS, KS), pl.ds(0, EMB)],
                    rows_v.at[nb],
                    hsems[nb],
                )

            pltpu.sync_copy(rows_v.at[b], acc_sh.at[idx_v.at[j]], add=True)
        return carry

    lax.fori_loop(0, NCHS // 2, outer, 0)
    plsc.subcore_barrier()
    pltpu.sync_copy(
        acc_sh.at[pl.ds(s * ROWS_PER_TILE, ROWS_PER_TILE)],
        out_hbm.at[c, pl.ds(s * ROWS_PER_TILE, ROWS_PER_TILE)],
    )


_scatter = pl.kernel(
    _scatter_body,
    out_type=jax.ShapeDtypeStruct((NUM_CORES, N_PAD, EMB), jnp.float32),
    mesh=plsc.VectorSubcoreMesh(core_axis_name="c", subcore_axis_name="s"),
    scratch_types=[
        pltpu.VMEM((NCHS, KS), jnp.int32),
        pltpu.VMEM((2, KS, EMB), jnp.float32),
        pltpu.VMEM_SHARED((N_PAD, EMB), jnp.float32),
        pltpu.SemaphoreType.DMA,
        pltpu.SemaphoreType.DMA,
    ],
    compiler_params=pltpu.CompilerParams(use_tc_tiling_on_sc=False),
)


def _edge_mlp_body(xs_ref, ea_ref, w1at_ref, w1ab_ref, b1a_ref, w1bx_ref, b1bx_ref, h_ref):
    h1 = jnp.maximum(
        jnp.dot(xs_ref[...], w1at_ref[...], preferred_element_type=jnp.float32)
        + jnp.dot(ea_ref[...], w1ab_ref[...], preferred_element_type=jnp.float32)
        + b1a_ref[...],
        0.0,
    )
    h_ref[...] = jnp.maximum(
        jnp.dot(h1, w1bx_ref[...], preferred_element_type=jnp.float32) + b1bx_ref[...],
        0.0,
    )


def _node_mlp_body(x_ref, agg_ref, w2at_ref, w2ab_ref, b2a_ref, w2b_ref, b2b_ref, out_ref):
    agg = agg_ref[0] + agg_ref[1]
    h = jnp.maximum(
        jnp.dot(x_ref[...], w2at_ref[...], preferred_element_type=jnp.float32)
        + jnp.dot(agg, w2ab_ref[...], preferred_element_type=jnp.float32)
        + b2a_ref[...],
        0.0,
    )
    out_ref[...] = jnp.maximum(
        jnp.dot(h, w2b_ref[...], preferred_element_type=jnp.float32) + b2b_ref[...],
        0.0,
    )


def _full_spec(shape):
    return pl.BlockSpec(shape, lambda i: (0,) * len(shape))


def kernel(x, edge_index, edge_attr, u, batch, W1a, b1a, W1b, b1b, W2a, b2a, W2b, b2b):
    senders = edge_index[0]
    receivers2 = edge_index[1].reshape(N_EDGES // KS, KS)
    w1a_top, w1a_bot = W1a[:2 * EMB], W1a[2 * EMB:]
    w2a_top, w2a_bot = W2a[:2 * EMB], W2a[2 * EMB:]
    w1b_x = jnp.concatenate([W1b, jnp.zeros((EMB, EMB), jnp.float32)], axis=1)
    b1b_x = jnp.concatenate([b1b, jnp.zeros((EMB,), jnp.float32)]).reshape(1, 2 * EMB)
    b1a2 = b1a.reshape(1, EMB)
    b2a2 = b2a.reshape(1, EMB)
    b2b2 = b2b.reshape(1, EMB)

    xs = _gather(x, senders)

    h = pl.pallas_call(
        _edge_mlp_body,
        grid=(N_EDGES // BE,),
        in_specs=[
            pl.BlockSpec((BE, 2 * EMB), lambda i: (i, 0)),
            pl.BlockSpec((BE, EMB), lambda i: (i, 0)),
            _full_spec((2 * EMB, EMB)),
            _full_spec((EMB, EMB)),
            _full_spec((1, EMB)),
            _full_spec((EMB, 2 * EMB)),
            _full_spec((1, 2 * EMB)),
        ],
        out_specs=pl.BlockSpec((BE, 2 * EMB), lambda i: (i, 0)),
        out_shape=jax.ShapeDtypeStruct((N_EDGES, 2 * EMB), jnp.float32),
    )(xs, edge_attr, w1a_top, w1a_bot, b1a2, w1b_x, b1b_x)

    agg2 = _scatter(h, receivers2)

    out = pl.pallas_call(
        _node_mlp_body,
        grid=(N_NODES // BN,),
        in_specs=[
            pl.BlockSpec((BN, 2 * EMB), lambda i: (i, 0)),
            pl.BlockSpec((NUM_CORES, BN, EMB), lambda i: (0, i, 0)),
            _full_spec((2 * EMB, EMB)),
            _full_spec((EMB, EMB)),
            _full_spec((1, EMB)),
            _full_spec((EMB, EMB)),
            _full_spec((1, EMB)),
        ],
        out_specs=pl.BlockSpec((BN, EMB), lambda i: (i, 0)),
        out_shape=jax.ShapeDtypeStruct((N_NODES, EMB), jnp.float32),
    )(x, agg2, w2a_top, w2a_bot, b2a2, W2b, b2b2)
    return out
